# baseline (speedup 1.0000x reference)
import functools

import jax
import jax.numpy as jnp
import numpy as np
from jax import lax
from jax.experimental import pallas as pl
from jax.experimental.pallas import tpu as pltpu
from jax.experimental.pallas import tpu_sc as plsc

SCALE = 0.25
PH = PW = 7
S = 2
C = 64
RP = 1024
LANES = 896
BINS = PH * PW
NR = RP * BINS
NW = 32
CHUNK_ROWS = 8
CHUNKS = NR // (NW * CHUNK_ROWS)


def _lane_consts():
    l = np.arange(LANES)
    p = l // 16
    q = l % 16
    py, px = p // PW, p % PW
    s, k = q // 4, q % 4
    offy = (s // S + 0.5) / S
    offx = (s % S + 0.5) / S
    ay = (py + offy).astype(np.float32)
    ax = (px + offx).astype(np.float32)
    ky = (k // 2).astype(np.float32)
    kx = (k % 2).astype(np.float32)
    valid = (p < BINS).astype(np.float32)
    return tuple(a.reshape(1, LANES) for a in (ay, ax, ky, kx, valid))


_AY, _AX, _KY, _KX, _VALID = _lane_consts()


def _coords_body(rois_ref, ay_ref, ax_ref, ky_ref, kx_ref, valid_ref,
                 idx1_ref, w1_ref, idx2_ref, w2_ref):
    r = rois_ref[...]
    b = r[:, 0:1].astype(jnp.int32)
    x1 = r[:, 1:2] * SCALE
    y1 = r[:, 2:3] * SCALE
    x2 = r[:, 3:4] * SCALE
    y2 = r[:, 4:5] * SCALE
    bw = jnp.maximum(x2 - x1, 1.0) * (1.0 / PW)
    bh = jnp.maximum(y2 - y1, 1.0) * (1.0 / PH)
    ay, ax = ay_ref[...], ax_ref[...]
    ky, kx = ky_ref[...], kx_ref[...]
    valid = valid_ref[...]
    kyi = ky.astype(jnp.int32)
    kxi = kx.astype(jnp.int32)
    y = y1 + ay * bh
    x = x1 + ax * bw

    def stage(H, W, idx_ref, w_ref):
        yc = jnp.clip(y, 0.0, float(H - 1))
        xc = jnp.clip(x, 0.0, float(W - 1))
        y0f = jnp.floor(yc)
        x0f = jnp.floor(xc)
        ly = yc - y0f
        lx = xc - x0f
        wy = ky * ly + (1.0 - ky) * (1.0 - ly)
        wx = kx * lx + (1.0 - kx) * (1.0 - lx)
        y0 = y0f.astype(jnp.int32)
        x0 = x0f.astype(jnp.int32)
        yi = jnp.minimum(y0 + kyi, H - 1)
        xi = jnp.minimum(x0 + kxi, W - 1)
        idx_ref[...] = b * (H * W) + yi * W + xi
        w_ref[...] = wy * wx * (1.0 / (S * S)) * valid

    stage(200, 200, idx1_ref, w1_ref)
    stage(PH, PW, idx2_ref, w2_ref)


def _coords(rois_p):
    grid = RP // 8
    cspec = pl.BlockSpec((1, LANES), lambda i: (0, 0))
    ospec = pl.BlockSpec((8, LANES), lambda i: (i, 0))
    return pl.pallas_call(
        _coords_body,
        grid=(grid,),
        in_specs=[pl.BlockSpec((8, 8), lambda i: (i, 0))] + [cspec] * 5,
        out_specs=[ospec] * 4,
        out_shape=[
            jax.ShapeDtypeStruct((RP, LANES), jnp.int32),
            jax.ShapeDtypeStruct((RP, LANES), jnp.float32),
            jax.ShapeDtypeStruct((RP, LANES), jnp.int32),
            jax.ShapeDtypeStruct((RP, LANES), jnp.float32),
        ],
    )(rois_p, jnp.asarray(_AY), jnp.asarray(_AX), jnp.asarray(_KY),
      jnp.asarray(_KX), jnp.asarray(_VALID))


def _gather_sum_kernel(table_hbm, idx_hbm, w_hbm, out_hbm,
                       idx_v, w_v, g_a, g_b, ob, sem_a, sem_b):
    nc = 2
    wid = lax.axis_index("s") * nc + lax.axis_index("c")
    pltpu.sync_copy(idx_hbm.at[wid], idx_v)
    pltpu.sync_copy(w_hbm.at[wid], w_v)
    row0 = wid * (CHUNKS * CHUNK_ROWS)

    def compute(j, g, buf):
        for r in range(CHUNK_ROWS):
            acc = [jnp.zeros((16,), jnp.float32) for _ in range(4)]
            wrow = buf[j, pl.ds(r * 16, 16)]
            for q in range(16):
                wv = jnp.full((16,), wrow[q], jnp.float32)
                for c in range(4):
                    acc[c] = acc[c] + wv * g[r * 16 + q, pl.ds(c * 16, 16)]
            for c in range(4):
                ob[r, pl.ds(c * 16, 16)] = acc[c]
        pltpu.sync_copy(ob, out_hbm.at[pl.ds(row0 + j * CHUNK_ROWS, CHUNK_ROWS)])

    pltpu.async_copy(table_hbm.at[idx_v.at[0]], g_a, sem_a)

    def body(jj, carry):
        j = jj * 2
        pltpu.async_copy(table_hbm.at[idx_v.at[j + 1]], g_b, sem_b)
        pltpu.make_async_copy(table_hbm.at[idx_v.at[j]], g_a, sem_a).wait()
        compute(j, g_a, w_v)

        @pl.when(jj < CHUNKS // 2 - 1)
        def _():
            pltpu.async_copy(table_hbm.at[idx_v.at[j + 2]], g_a, sem_a)

        pltpu.make_async_copy(table_hbm.at[idx_v.at[j + 1]], g_b, sem_b).wait()
        compute(j + 1, g_b, w_v)
        return carry

    lax.fori_loop(0, CHUNKS // 2, body, 0)


def _gather_sum(table, idx, w):
    mesh = plsc.VectorSubcoreMesh(core_axis_name="c", subcore_axis_name="s")
    kfn = functools.partial(
        pl.kernel,
        mesh=mesh,
        compiler_params=pltpu.CompilerParams(use_tc_tiling_on_sc=False),
        out_type=jax.ShapeDtypeStruct((NR, C), jnp.float32),
        scratch_types=[
            pltpu.VMEM((CHUNKS, 128), jnp.int32),
            pltpu.VMEM((CHUNKS, 128), jnp.float32),
            pltpu.VMEM((128, C), jnp.float32),
            pltpu.VMEM((128, C), jnp.float32),
            pltpu.VMEM((CHUNK_ROWS, C), jnp.float32),
            pltpu.SemaphoreType.DMA,
            pltpu.SemaphoreType.DMA,
        ],
    )(_gather_sum_kernel)
    return kfn(table, idx, w)


def _pack(a):
    a = a.reshape(RP, LANES // 16, 16)[:, :BINS, :]
    return a.reshape(NW, CHUNKS, 128)


def kernel(input, rois):
    feat = jnp.transpose(input, (0, 2, 3, 1)).reshape(-1, C)
    rois_p = jnp.zeros((RP, 8), jnp.float32).at[:rois.shape[0], :5].set(rois)
    idx1, w1, idx2, w2 = _coords(rois_p)
    o1 = _gather_sum(feat, _pack(idx1), _pack(w1))
    t2 = o1[: 2 * BINS]
    o2 = _gather_sum(t2, _pack(idx2), _pack(w2))
    out = o2[: rois.shape[0] * BINS].reshape(-1, BINS, C)
    return jnp.transpose(out, (0, 2, 1)).reshape(-1, C, PH, PW)


# speedup vs baseline: 1.0036x; 1.0036x over previous
import functools

import jax
import jax.numpy as jnp
import numpy as np
from jax import lax
from jax.experimental import pallas as pl
from jax.experimental.pallas import tpu as pltpu
from jax.experimental.pallas import tpu_sc as plsc

SCALE = 0.25
PH = PW = 7
S = 2
C = 64
RP = 1024
LANES = 896
BINS = PH * PW
NR = RP * BINS
NW = 32
CHUNK_ROWS = 8
CHUNKS = NR // (NW * CHUNK_ROWS)


def _lane_consts():
    l = np.arange(LANES)
    p = l // 16
    q = l % 16
    py, px = p // PW, p % PW
    ygrp, xgrp = q // 4, q % 4
    sy, ky_i = ygrp // 2, ygrp % 2
    sx, kx_i = xgrp // 2, xgrp % 2
    offy = (sy + 0.5) / S
    offx = (sx + 0.5) / S
    ay = (py + offy).astype(np.float32)
    ax = (px + offx).astype(np.float32)
    ky = ky_i.astype(np.float32)
    kx = kx_i.astype(np.float32)
    valid = (p < BINS).astype(np.float32)
    return tuple(a.reshape(1, LANES) for a in (ay, ax, ky, kx, valid))


_AY, _AX, _KY, _KX, _VALID = _lane_consts()


def _coords_body(rois_ref, ay_ref, ax_ref, ky_ref, kx_ref, valid_ref,
                 idx1_ref, w1_ref, idx2_ref, w2_ref):
    r = rois_ref[...]
    b = r[:, 0:1].astype(jnp.int32)
    x1 = r[:, 1:2] * SCALE
    y1 = r[:, 2:3] * SCALE
    x2 = r[:, 3:4] * SCALE
    y2 = r[:, 4:5] * SCALE
    bw = jnp.maximum(x2 - x1, 1.0) * (1.0 / PW)
    bh = jnp.maximum(y2 - y1, 1.0) * (1.0 / PH)
    ay, ax = ay_ref[...], ax_ref[...]
    ky, kx = ky_ref[...], kx_ref[...]
    valid = valid_ref[...]
    kyi = ky.astype(jnp.int32)
    kxi = kx.astype(jnp.int32)
    y = y1 + ay * bh
    x = x1 + ax * bw

    def stage(H, W, idx_ref, w_ref):
        yc = jnp.clip(y, 0.0, float(H - 1))
        xc = jnp.clip(x, 0.0, float(W - 1))
        y0f = jnp.floor(yc)
        x0f = jnp.floor(xc)
        ly = yc - y0f
        lx = xc - x0f
        wy = ky * ly + (1.0 - ky) * (1.0 - ly)
        wx = kx * lx + (1.0 - kx) * (1.0 - lx)
        y0 = y0f.astype(jnp.int32)
        x0 = x0f.astype(jnp.int32)
        yi = jnp.minimum(y0 + kyi, H - 1)
        xi = jnp.minimum(x0 + kxi, W - 1)
        idx_ref[...] = b * (H * W) + yi * W + xi
        w_ref[...] = wy * wx * (1.0 / (S * S)) * valid

    stage(200, 200, idx1_ref, w1_ref)
    stage(PH, PW, idx2_ref, w2_ref)


def _coords(rois_p):
    grid = RP // 8
    cspec = pl.BlockSpec((1, LANES), lambda i: (0, 0))
    ospec = pl.BlockSpec((8, LANES), lambda i: (i, 0))
    return pl.pallas_call(
        _coords_body,
        grid=(grid,),
        in_specs=[pl.BlockSpec((8, 8), lambda i: (i, 0))] + [cspec] * 5,
        out_specs=[ospec] * 4,
        out_shape=[
            jax.ShapeDtypeStruct((RP, LANES), jnp.int32),
            jax.ShapeDtypeStruct((RP, LANES), jnp.float32),
            jax.ShapeDtypeStruct((RP, LANES), jnp.int32),
            jax.ShapeDtypeStruct((RP, LANES), jnp.float32),
        ],
    )(rois_p, jnp.asarray(_AY), jnp.asarray(_AX), jnp.asarray(_KY),
      jnp.asarray(_KX), jnp.asarray(_VALID))


NSUB = 4
SUBROWS = CHUNK_ROWS * 16 // NSUB


def _gather_sum_kernel(table_hbm, idx_hbm, w_hbm, out_hbm,
                       idx_v, w_v, g_a, g_b, ob, sem_a, sem_b):
    nc = 2
    wid = lax.axis_index("s") * nc + lax.axis_index("c")
    pltpu.sync_copy(idx_hbm.at[wid], idx_v)
    pltpu.sync_copy(w_hbm.at[wid], w_v)
    row0 = wid * (CHUNKS * CHUNK_ROWS)

    def fire(j, g, sem):
        for s in range(NSUB):
            pltpu.async_copy(table_hbm.at[idx_v.at[j * NSUB + s]],
                             g.at[pl.ds(s * SUBROWS, SUBROWS)], sem)

    def drain(j, g, sem):
        for s in range(NSUB):
            pltpu.make_async_copy(table_hbm.at[idx_v.at[j * NSUB + s]],
                                  g.at[pl.ds(s * SUBROWS, SUBROWS)], sem).wait()

    def compute(j, g, obase):
        for r in range(CHUNK_ROWS):
            acc = [jnp.zeros((16,), jnp.float32) for _ in range(4)]
            wrow = w_v[j, pl.ds(r * 16, 16)]
            for q in range(16):
                wv = jnp.full((16,), wrow[q], jnp.float32)
                for c in range(4):
                    acc[c] = acc[c] + wv * g[r * 16 + q, pl.ds(c * 16, 16)]
            for c in range(4):
                ob[obase + r, pl.ds(c * 16, 16)] = acc[c]

    fire(0, g_a, sem_a)

    def body(jj, carry):
        j = jj * 2
        fire(j + 1, g_b, sem_b)
        drain(j, g_a, sem_a)
        compute(j, g_a, 0)

        @pl.when(jj < CHUNKS // 2 - 1)
        def _():
            fire(j + 2, g_a, sem_a)

        drain(j + 1, g_b, sem_b)
        compute(j + 1, g_b, CHUNK_ROWS)
        pltpu.sync_copy(ob, out_hbm.at[pl.ds(row0 + j * CHUNK_ROWS,
                                             2 * CHUNK_ROWS)])
        return carry

    lax.fori_loop(0, CHUNKS // 2, body, 0)


def _gather_sum(table, idx, w):
    mesh = plsc.VectorSubcoreMesh(core_axis_name="c", subcore_axis_name="s")
    kfn = functools.partial(
        pl.kernel,
        mesh=mesh,
        compiler_params=pltpu.CompilerParams(use_tc_tiling_on_sc=False),
        out_type=jax.ShapeDtypeStruct((NR, C), jnp.float32),
        scratch_types=[
            pltpu.VMEM((CHUNKS * NSUB, SUBROWS), jnp.int32),
            pltpu.VMEM((CHUNKS, 128), jnp.float32),
            pltpu.VMEM((128, C), jnp.float32),
            pltpu.VMEM((128, C), jnp.float32),
            pltpu.VMEM((2 * CHUNK_ROWS, C), jnp.float32),
            pltpu.SemaphoreType.DMA,
            pltpu.SemaphoreType.DMA,
        ],
    )(_gather_sum_kernel)
    return kfn(table, idx, w)


def _pack(a, cols=128):
    a = a.reshape(RP, LANES // 16, 16)[:, :BINS, :]
    return a.reshape(NW, (CHUNKS * 128) // cols, cols)


def kernel(input, rois):
    feat = jnp.transpose(input, (0, 2, 3, 1)).reshape(-1, C)
    rois_p = jnp.zeros((RP, 8), jnp.float32).at[:rois.shape[0], :5].set(rois)
    idx1, w1, idx2, w2 = _coords(rois_p)
    o1 = _gather_sum(feat, _pack(idx1, SUBROWS), _pack(w1))
    t2 = o1[: 2 * BINS]
    o2 = _gather_sum(t2, _pack(idx2, SUBROWS), _pack(w2))
    out = o2[: rois.shape[0] * BINS].reshape(-1, BINS, C)
    return jnp.transpose(out, (0, 2, 1)).reshape(-1, C, PH, PW)


# speedup vs baseline: 1.0044x; 1.0008x over previous
import functools

import jax
import jax.numpy as jnp
import numpy as np
from jax import lax
from jax.experimental import pallas as pl
from jax.experimental.pallas import tpu as pltpu
from jax.experimental.pallas import tpu_sc as plsc

SCALE = 0.25
PH = PW = 7
S = 2
C = 64
RP = 1024
LANES = 896
BINS = PH * PW
NR = RP * BINS
NW = 32
CHUNK_ROWS = 8
CHUNKS = NR // (NW * CHUNK_ROWS)


def _lane_consts():
    l = np.arange(LANES)
    p = l // 16
    q = l % 16
    py, px = p // PW, p % PW
    ygrp, xgrp = q // 4, q % 4
    sy, ky_i = ygrp // 2, ygrp % 2
    sx, kx_i = xgrp // 2, xgrp % 2
    offy = (sy + 0.5) / S
    offx = (sx + 0.5) / S
    ay = (py + offy).astype(np.float32)
    ax = (px + offx).astype(np.float32)
    ky = ky_i.astype(np.float32)
    kx = kx_i.astype(np.float32)
    valid = (p < BINS).astype(np.float32)
    return tuple(a.reshape(1, LANES) for a in (ay, ax, ky, kx, valid))


_AY, _AX, _KY, _KX, _VALID = _lane_consts()


def _coords_body(rois_ref, ay_ref, ax_ref, ky_ref, kx_ref, valid_ref,
                 idx1_ref, w1_ref, idx2_ref, w2_ref):
    r = rois_ref[...]
    b = r[:, 0:1].astype(jnp.int32)
    x1 = r[:, 1:2] * SCALE
    y1 = r[:, 2:3] * SCALE
    x2 = r[:, 3:4] * SCALE
    y2 = r[:, 4:5] * SCALE
    bw = jnp.maximum(x2 - x1, 1.0) * (1.0 / PW)
    bh = jnp.maximum(y2 - y1, 1.0) * (1.0 / PH)
    ay, ax = ay_ref[...], ax_ref[...]
    ky, kx = ky_ref[...], kx_ref[...]
    valid = valid_ref[...]
    kyi = ky.astype(jnp.int32)
    kxi = kx.astype(jnp.int32)
    y = y1 + ay * bh
    x = x1 + ax * bw

    def stage(H, W, idx_ref, w_ref):
        yc = jnp.clip(y, 0.0, float(H - 1))
        xc = jnp.clip(x, 0.0, float(W - 1))
        y0f = jnp.floor(yc)
        x0f = jnp.floor(xc)
        ly = yc - y0f
        lx = xc - x0f
        wy = ky * ly + (1.0 - ky) * (1.0 - ly)
        wx = kx * lx + (1.0 - kx) * (1.0 - lx)
        y0 = y0f.astype(jnp.int32)
        x0 = x0f.astype(jnp.int32)
        yi = jnp.minimum(y0 + kyi, H - 1)
        xi = jnp.minimum(x0 + kxi, W - 1)
        idx_ref[...] = b * (H * W) + yi * W + xi
        w_ref[...] = wy * wx * (1.0 / (S * S)) * valid

    stage(200, 200, idx1_ref, w1_ref)
    stage(PH, PW, idx2_ref, w2_ref)


def _coords(rois_p):
    grid = RP // 8
    cspec = pl.BlockSpec((1, LANES), lambda i: (0, 0))
    ospec = pl.BlockSpec((8, LANES), lambda i: (i, 0))
    return pl.pallas_call(
        _coords_body,
        grid=(grid,),
        in_specs=[pl.BlockSpec((8, 8), lambda i: (i, 0))] + [cspec] * 5,
        out_specs=[ospec] * 4,
        out_shape=[
            jax.ShapeDtypeStruct((RP, LANES), jnp.int32),
            jax.ShapeDtypeStruct((RP, LANES), jnp.float32),
            jax.ShapeDtypeStruct((RP, LANES), jnp.int32),
            jax.ShapeDtypeStruct((RP, LANES), jnp.float32),
        ],
    )(rois_p, jnp.asarray(_AY), jnp.asarray(_AX), jnp.asarray(_KY),
      jnp.asarray(_KX), jnp.asarray(_VALID))


NSUB = 4
SUBROWS = CHUNK_ROWS * 16 // NSUB
NPASS = 4
PC = C // NPASS
TROWS = 2 * 200 * 200


def _gather_sum_cp_kernel(table_hbm, idx_hbm, w_hbm,
                          o0, o1, o2, o3,
                          spm, idx_v, w_v, g_a, g_b, ob, sem_a, sem_b):
    outs = (o0, o1, o2, o3)
    nc = 2
    hchunks = CHUNKS // 2
    sid = lax.axis_index("s")
    wid = sid * nc + lax.axis_index("c")
    row0 = wid * (CHUNKS * CHUNK_ROWS)
    stage_rows = TROWS // 16

    for p in range(NPASS):
        if p:
            plsc.subcore_barrier()
        pltpu.sync_copy(table_hbm.at[p, pl.ds(sid * stage_rows, stage_rows)],
                        spm.at[pl.ds(sid * stage_rows, stage_rows)])
        plsc.subcore_barrier()
        out = outs[p]

        def fire(j, g, sem):
            for s in range(NSUB):
                pltpu.async_copy(spm.at[idx_v.at[j * NSUB + s]],
                                 g.at[pl.ds(s * SUBROWS, SUBROWS)], sem)

        def drain(j, g, sem):
            for s in range(NSUB):
                pltpu.make_async_copy(spm.at[idx_v.at[j * NSUB + s]],
                                      g.at[pl.ds(s * SUBROWS, SUBROWS)],
                                      sem).wait()

        def compute(j, g, obase):
            for r in range(CHUNK_ROWS):
                acc = jnp.zeros((16,), jnp.float32)
                wrow = w_v[j, pl.ds(r * 16, 16)]
                for q in range(16):
                    wv = jnp.full((16,), wrow[q], jnp.float32)
                    acc = acc + wv * g[r * 16 + q, :]
                ob[obase + r, :] = acc

        for h in range(2):
            pltpu.sync_copy(idx_hbm.at[wid, pl.ds(h * hchunks * NSUB,
                                                  hchunks * NSUB)], idx_v)
            pltpu.sync_copy(w_hbm.at[wid, pl.ds(h * hchunks, hchunks)], w_v)
            hrow0 = row0 + h * hchunks * CHUNK_ROWS

            fire(0, g_a, sem_a)

            def body(jj, carry):
                j = jj * 2
                fire(j + 1, g_b, sem_b)
                drain(j, g_a, sem_a)
                compute(j, g_a, 0)

                @pl.when(jj < hchunks // 2 - 1)
                def _():
                    fire(j + 2, g_a, sem_a)

                drain(j + 1, g_b, sem_b)
                compute(j + 1, g_b, CHUNK_ROWS)
                pltpu.sync_copy(ob, out.at[pl.ds(hrow0 + j * CHUNK_ROWS,
                                                 2 * CHUNK_ROWS)])
                return carry

            lax.fori_loop(0, hchunks // 2, body, 0)


def _gather_sum_cp(table_cp, idx, w):
    mesh = plsc.VectorSubcoreMesh(core_axis_name="c", subcore_axis_name="s")
    oshape = jax.ShapeDtypeStruct((NR, PC), jnp.float32)
    kfn = functools.partial(
        pl.kernel,
        mesh=mesh,
        compiler_params=pltpu.CompilerParams(use_tc_tiling_on_sc=False),
        out_type=[oshape] * NPASS,
        scratch_types=[
            pltpu.VMEM_SHARED((TROWS, PC), jnp.float32),
            pltpu.VMEM((CHUNKS // 2 * NSUB, SUBROWS), jnp.int32),
            pltpu.VMEM((CHUNKS // 2, 128), jnp.float32),
            pltpu.VMEM((128, PC), jnp.float32),
            pltpu.VMEM((128, PC), jnp.float32),
            pltpu.VMEM((2 * CHUNK_ROWS, PC), jnp.float32),
            pltpu.SemaphoreType.DMA,
            pltpu.SemaphoreType.DMA,
        ],
    )(_gather_sum_cp_kernel)
    return kfn(table_cp, idx, w)


def _gather_sum_kernel(table_hbm, idx_hbm, w_hbm, out_hbm,
                       idx_v, w_v, g_a, g_b, ob, sem_a, sem_b):
    nc = 2
    wid = lax.axis_index("s") * nc + lax.axis_index("c")
    pltpu.sync_copy(idx_hbm.at[wid], idx_v)
    pltpu.sync_copy(w_hbm.at[wid], w_v)
    row0 = wid * (CHUNKS * CHUNK_ROWS)

    def fire(j, g, sem):
        for s in range(NSUB):
            pltpu.async_copy(table_hbm.at[idx_v.at[j * NSUB + s]],
                             g.at[pl.ds(s * SUBROWS, SUBROWS)], sem)

    def drain(j, g, sem):
        for s in range(NSUB):
            pltpu.make_async_copy(table_hbm.at[idx_v.at[j * NSUB + s]],
                                  g.at[pl.ds(s * SUBROWS, SUBROWS)], sem).wait()

    def compute(j, g, obase):
        for r in range(CHUNK_ROWS):
            acc = [jnp.zeros((16,), jnp.float32) for _ in range(4)]
            wrow = w_v[j, pl.ds(r * 16, 16)]
            for q in range(16):
                wv = jnp.full((16,), wrow[q], jnp.float32)
                for c in range(4):
                    acc[c] = acc[c] + wv * g[r * 16 + q, pl.ds(c * 16, 16)]
            for c in range(4):
                ob[obase + r, pl.ds(c * 16, 16)] = acc[c]

    fire(0, g_a, sem_a)

    def body(jj, carry):
        j = jj * 2
        fire(j + 1, g_b, sem_b)
        drain(j, g_a, sem_a)
        compute(j, g_a, 0)

        @pl.when(jj < CHUNKS // 2 - 1)
        def _():
            fire(j + 2, g_a, sem_a)

        drain(j + 1, g_b, sem_b)
        compute(j + 1, g_b, CHUNK_ROWS)
        pltpu.sync_copy(ob, out_hbm.at[pl.ds(row0 + j * CHUNK_ROWS,
                                             2 * CHUNK_ROWS)])
        return carry

    lax.fori_loop(0, CHUNKS // 2, body, 0)


def _gather_sum(table, idx, w):
    mesh = plsc.VectorSubcoreMesh(core_axis_name="c", subcore_axis_name="s")
    kfn = functools.partial(
        pl.kernel,
        mesh=mesh,
        compiler_params=pltpu.CompilerParams(use_tc_tiling_on_sc=False),
        out_type=jax.ShapeDtypeStruct((NR, C), jnp.float32),
        scratch_types=[
            pltpu.VMEM((CHUNKS * NSUB, SUBROWS), jnp.int32),
            pltpu.VMEM((CHUNKS, 128), jnp.float32),
            pltpu.VMEM((128, C), jnp.float32),
            pltpu.VMEM((128, C), jnp.float32),
            pltpu.VMEM((2 * CHUNK_ROWS, C), jnp.float32),
            pltpu.SemaphoreType.DMA,
            pltpu.SemaphoreType.DMA,
        ],
    )(_gather_sum_kernel)
    return kfn(table, idx, w)


def _pack(a, cols=128):
    a = a.reshape(RP, LANES // 16, 16)[:, :BINS, :]
    return a.reshape(NW, (CHUNKS * 128) // cols, cols)


def kernel(input, rois):
    feat = jnp.transpose(input, (0, 2, 3, 1)).reshape(-1, C)
    feat_cp = feat.reshape(TROWS, NPASS, PC).transpose(1, 0, 2)
    rois_p = jnp.zeros((RP, 8), jnp.float32).at[:rois.shape[0], :5].set(rois)
    idx1, w1, idx2, w2 = _coords(rois_p)
    o1_parts = _gather_sum_cp(feat_cp, _pack(idx1, SUBROWS), _pack(w1))
    o1 = jnp.concatenate(o1_parts, axis=1)
    t2 = o1[: 2 * BINS]
    o2 = _gather_sum(t2, _pack(idx2, SUBROWS), _pack(w2))
    out = o2[: rois.shape[0] * BINS].reshape(-1, BINS, C)
    return jnp.transpose(out, (0, 2, 1)).reshape(-1, C, PH, PW)


# speedup vs baseline: 11.5187x; 11.4682x over previous
import functools

import jax
import jax.numpy as jnp
import numpy as np
from jax import lax
from jax.experimental import pallas as pl
from jax.experimental.pallas import tpu as pltpu
from jax.experimental.pallas import tpu_sc as plsc

SCALE = 0.25
PH = PW = 7
S = 2
C = 64
RP = 1024
LANES = 896
BINS = PH * PW
NR = RP * BINS
NW = 32
CHUNK_ROWS = 8
CHUNKS = NR // (NW * CHUNK_ROWS)


def _lane_consts():
    l = np.arange(LANES)
    p = l // 16
    q = l % 16
    py, px = p // PW, p % PW
    ygrp, xgrp = q // 4, q % 4
    sy, ky_i = ygrp // 2, ygrp % 2
    sx, kx_i = xgrp // 2, xgrp % 2
    offy = (sy + 0.5) / S
    offx = (sx + 0.5) / S
    ay = (py + offy).astype(np.float32)
    ax = (px + offx).astype(np.float32)
    ky = ky_i.astype(np.float32)
    kx = kx_i.astype(np.float32)
    valid = (p < BINS).astype(np.float32)
    return tuple(a.reshape(1, LANES) for a in (ay, ax, ky, kx, valid))


_AY, _AX, _KY, _KX, _VALID = _lane_consts()


def _coords_body(rois_ref, ay_ref, ax_ref, ky_ref, kx_ref, valid_ref,
                 idx1_ref, w1_ref, idx2_ref, w2_ref):
    r = rois_ref[...]
    b = r[:, 0:1].astype(jnp.int32)
    x1 = r[:, 1:2] * SCALE
    y1 = r[:, 2:3] * SCALE
    x2 = r[:, 3:4] * SCALE
    y2 = r[:, 4:5] * SCALE
    bw = jnp.maximum(x2 - x1, 1.0) * (1.0 / PW)
    bh = jnp.maximum(y2 - y1, 1.0) * (1.0 / PH)
    ay, ax = ay_ref[...], ax_ref[...]
    ky, kx = ky_ref[...], kx_ref[...]
    valid = valid_ref[...]
    kyi = ky.astype(jnp.int32)
    kxi = kx.astype(jnp.int32)
    y = y1 + ay * bh
    x = x1 + ax * bw

    def stage(H, W, idx_ref, w_ref):
        yc = jnp.clip(y, 0.0, float(H - 1))
        xc = jnp.clip(x, 0.0, float(W - 1))
        y0f = jnp.floor(yc)
        x0f = jnp.floor(xc)
        ly = yc - y0f
        lx = xc - x0f
        wy = ky * ly + (1.0 - ky) * (1.0 - ly)
        wx = kx * lx + (1.0 - kx) * (1.0 - lx)
        y0 = y0f.astype(jnp.int32)
        x0 = x0f.astype(jnp.int32)
        yi = jnp.minimum(y0 + kyi, H - 1)
        xi = jnp.minimum(x0 + kxi, W - 1)
        idx_ref[...] = b * (H * W) + yi * W + xi
        w_ref[...] = wy * wx * (1.0 / (S * S)) * valid

    stage(200, 200, idx1_ref, w1_ref)
    stage(PH, PW, idx2_ref, w2_ref)


def _coords(rois_p):
    grid = RP // 8
    cspec = pl.BlockSpec((1, LANES), lambda i: (0, 0))
    ospec = pl.BlockSpec((8, LANES), lambda i: (i, 0))
    return pl.pallas_call(
        _coords_body,
        grid=(grid,),
        in_specs=[pl.BlockSpec((8, 8), lambda i: (i, 0))] + [cspec] * 5,
        out_specs=[ospec] * 4,
        out_shape=[
            jax.ShapeDtypeStruct((RP, LANES), jnp.int32),
            jax.ShapeDtypeStruct((RP, LANES), jnp.float32),
            jax.ShapeDtypeStruct((RP, LANES), jnp.int32),
            jax.ShapeDtypeStruct((RP, LANES), jnp.float32),
        ],
    )(rois_p, jnp.asarray(_AY), jnp.asarray(_AX), jnp.asarray(_KY),
      jnp.asarray(_KX), jnp.asarray(_VALID))


NSUB = 4
SUBROWS = CHUNK_ROWS * 16 // NSUB
NPASS = 4
PC = C // NPASS
TROWS = 2 * 200 * 200


def _gather_sum_cp_kernel(table_hbm, idx_hbm, w_hbm,
                          o0, o1, o2, o3,
                          spm, idx_v, w_v, g_a, g_b, ovm, sem_a, sem_b):
    outs = (o0, o1, o2, o3)
    nc = 2
    hchunks = CHUNKS // 2
    sid = lax.axis_index("s")
    wid = sid * nc + lax.axis_index("c")
    row0 = wid * (CHUNKS * CHUNK_ROWS)
    stage_rows = TROWS // 16

    for p in range(NPASS):
        if p:
            plsc.subcore_barrier()
        pltpu.sync_copy(table_hbm.at[p, pl.ds(sid * stage_rows, stage_rows)],
                        spm.at[pl.ds(sid * stage_rows, stage_rows)])
        plsc.subcore_barrier()
        out = outs[p]

        def fire(j, g, sem):
            for s in range(NSUB):
                pltpu.async_copy(spm.at[idx_v.at[j * NSUB + s]],
                                 g.at[pl.ds(s * SUBROWS, SUBROWS)], sem)

        def drain(j, g, sem):
            for s in range(NSUB):
                pltpu.make_async_copy(spm.at[idx_v.at[j * NSUB + s]],
                                      g.at[pl.ds(s * SUBROWS, SUBROWS)],
                                      sem).wait()

        def compute(j, g):
            for r in range(CHUNK_ROWS):
                acc = jnp.zeros((16,), jnp.float32)
                wrow = w_v[j, pl.ds(r * 16, 16)]
                for q in range(16):
                    wv = jnp.full((16,), wrow[q], jnp.float32)
                    acc = acc + wv * g[r * 16 + q, :]
                ovm[pl.ds((j * CHUNK_ROWS + r) * PC, PC)] = acc

        for h in range(2):
            pltpu.sync_copy(idx_hbm.at[wid, pl.ds(h * hchunks * NSUB,
                                                  hchunks * NSUB)], idx_v)
            pltpu.sync_copy(w_hbm.at[wid, pl.ds(h * hchunks, hchunks)], w_v)

            fire(0, g_a, sem_a)

            def body(jj, carry):
                j = jj * 2
                fire(j + 1, g_b, sem_b)
                drain(j, g_a, sem_a)
                compute(j, g_a)

                @pl.when(jj < hchunks // 2 - 1)
                def _():
                    fire(j + 2, g_a, sem_a)

                drain(j + 1, g_b, sem_b)
                compute(j + 1, g_b)
                return carry

            lax.fori_loop(0, hchunks // 2, body, 0)
            pltpu.sync_copy(ovm, out.at[pl.ds(
                (row0 + h * hchunks * CHUNK_ROWS) * PC,
                hchunks * CHUNK_ROWS * PC)])


def _gather_sum_cp(table_cp, idx, w):
    mesh = plsc.VectorSubcoreMesh(core_axis_name="c", subcore_axis_name="s")
    oshape = jax.ShapeDtypeStruct((NR * PC,), jnp.float32)
    kfn = functools.partial(
        pl.kernel,
        mesh=mesh,
        compiler_params=pltpu.CompilerParams(use_tc_tiling_on_sc=False),
        out_type=[oshape] * NPASS,
        scratch_types=[
            pltpu.VMEM_SHARED((TROWS, PC), jnp.float32),
            pltpu.VMEM((CHUNKS // 2 * NSUB, SUBROWS), jnp.int32),
            pltpu.VMEM((CHUNKS // 2, 128), jnp.float32),
            pltpu.VMEM((128, PC), jnp.float32),
            pltpu.VMEM((128, PC), jnp.float32),
            pltpu.VMEM((CHUNKS // 2 * CHUNK_ROWS * PC,), jnp.float32),
            pltpu.SemaphoreType.DMA,
            pltpu.SemaphoreType.DMA,
        ],
    )(_gather_sum_cp_kernel)
    return kfn(table_cp, idx, w)


def _stage2_kernel(table_hbm, idx_hbm, w_hbm, out_hbm, tb, idx_v, w_v, ovm):
    nc = 2
    hchunks = CHUNKS // 2
    wid = lax.axis_index("s") * nc + lax.axis_index("c")
    pltpu.sync_copy(table_hbm, tb)
    row0 = wid * (CHUNKS * CHUNK_ROWS)

    for h in range(2):
        pltpu.sync_copy(idx_hbm.at[wid, pl.ds(h * hchunks * NSUB,
                                              hchunks * NSUB)], idx_v)
        pltpu.sync_copy(w_hbm.at[wid, pl.ds(h * hchunks, hchunks)], w_v)

        def body(j, carry):
            for r in range(CHUNK_ROWS):
                irow = idx_v[j * NSUB + r // 2, pl.ds((r % 2) * 16, 16)]
                wrow = w_v[j, pl.ds(r * 16, 16)]
                acc = [jnp.zeros((16,), jnp.float32) for _ in range(4)]
                for q in range(16):
                    base = irow[q] * C
                    wv = jnp.full((16,), wrow[q], jnp.float32)
                    for c in range(4):
                        acc[c] = acc[c] + wv * tb[pl.ds(base + c * 16, 16)]
                for c in range(4):
                    ovm[pl.ds((j * CHUNK_ROWS + r) * C + c * 16, 16)] = acc[c]
            return carry

        lax.fori_loop(0, hchunks, body, 0)
        pltpu.sync_copy(ovm, out_hbm.at[pl.ds(
            (row0 + h * hchunks * CHUNK_ROWS) * C,
            hchunks * CHUNK_ROWS * C)])


def _stage2(table_flat, idx, w):
    mesh = plsc.VectorSubcoreMesh(core_axis_name="c", subcore_axis_name="s")
    hrows = (CHUNKS // 2) * CHUNK_ROWS
    kfn = functools.partial(
        pl.kernel,
        mesh=mesh,
        compiler_params=pltpu.CompilerParams(use_tc_tiling_on_sc=False),
        out_type=jax.ShapeDtypeStruct((NR * C,), jnp.float32),
        scratch_types=[
            pltpu.VMEM((2 * BINS * C,), jnp.float32),
            pltpu.VMEM((CHUNKS // 2 * NSUB, SUBROWS), jnp.int32),
            pltpu.VMEM((CHUNKS // 2, 128), jnp.float32),
            pltpu.VMEM((hrows * C,), jnp.float32),
        ],
    )(_stage2_kernel)
    return kfn(table_flat, idx, w)


def _pack(a, cols=128):
    a = a.reshape(RP, LANES // 16, 16)[:, :BINS, :]
    return a.reshape(NW, (CHUNKS * 128) // cols, cols)


def kernel(input, rois):
    feat = jnp.transpose(input, (0, 2, 3, 1)).reshape(-1, C)
    feat_cp = feat.reshape(TROWS, NPASS, PC).transpose(1, 0, 2)
    rois_p = jnp.zeros((RP, 8), jnp.float32).at[:rois.shape[0], :5].set(rois)
    idx1, w1, idx2, w2 = _coords(rois_p)
    o1_parts = _gather_sum_cp(feat_cp, _pack(idx1, SUBROWS), _pack(w1))
    o1 = jnp.concatenate([o.reshape(NR, PC) for o in o1_parts], axis=1)
    t2 = o1[: 2 * BINS].reshape(-1)
    o2 = _stage2(t2, _pack(idx2, SUBROWS), _pack(w2)).reshape(NR, C)
    out = o2[: rois.shape[0] * BINS].reshape(-1, BINS, C)
    return jnp.transpose(out, (0, 2, 1)).reshape(-1, C, PH, PW)


# speedup vs baseline: 14.2498x; 1.2371x over previous
import functools

import jax
import jax.numpy as jnp
import numpy as np
from jax import lax
from jax.experimental import pallas as pl
from jax.experimental.pallas import tpu as pltpu
from jax.experimental.pallas import tpu_sc as plsc

SCALE = 0.25
PH = PW = 7
S = 2
C = 64
RP = 1024
LANES = 784
BINS = PH * PW
NR = RP * BINS
NW = 32
CHUNK_ROWS = 8
CHUNKS = NR // (NW * CHUNK_ROWS)


def _lane_consts():
    l = np.arange(LANES)
    p = l // 16
    q = l % 16
    py, px = p // PW, p % PW
    ygrp, xgrp = q // 4, q % 4
    sy, ky_i = ygrp // 2, ygrp % 2
    sx, kx_i = xgrp // 2, xgrp % 2
    offy = (sy + 0.5) / S
    offx = (sx + 0.5) / S
    ay = (py + offy).astype(np.float32)
    ax = (px + offx).astype(np.float32)
    ky = ky_i.astype(np.float32)
    kx = kx_i.astype(np.float32)
    return tuple(a.reshape(1, LANES) for a in (ay, ax, ky, kx))


_AY, _AX, _KY, _KX = _lane_consts()


def _coords_body(rois_ref, ay_ref, ax_ref, ky_ref, kx_ref,
                 idx1_ref, w1_ref, idx2_ref, w2_ref):
    r = rois_ref[...]
    b = r[:, 0:1].astype(jnp.int32)
    x1 = r[:, 1:2] * SCALE
    y1 = r[:, 2:3] * SCALE
    x2 = r[:, 3:4] * SCALE
    y2 = r[:, 4:5] * SCALE
    bw = jnp.maximum(x2 - x1, 1.0) * (1.0 / PW)
    bh = jnp.maximum(y2 - y1, 1.0) * (1.0 / PH)
    ay, ax = ay_ref[...], ax_ref[...]
    ky, kx = ky_ref[...], kx_ref[...]
    kyi = ky.astype(jnp.int32)
    kxi = kx.astype(jnp.int32)
    y = y1 + ay * bh
    x = x1 + ax * bw

    def stage(H, W, idx_ref, w_ref):
        yc = jnp.clip(y, 0.0, float(H - 1))
        xc = jnp.clip(x, 0.0, float(W - 1))
        y0f = jnp.floor(yc)
        x0f = jnp.floor(xc)
        ly = yc - y0f
        lx = xc - x0f
        wy = ky * ly + (1.0 - ky) * (1.0 - ly)
        wx = kx * lx + (1.0 - kx) * (1.0 - lx)
        y0 = y0f.astype(jnp.int32)
        x0 = x0f.astype(jnp.int32)
        yi = jnp.minimum(y0 + kyi, H - 1)
        xi = jnp.minimum(x0 + kxi, W - 1)
        idx_ref[...] = b * (H * W) + yi * W + xi
        w_ref[...] = wy * wx * (1.0 / (S * S))

    stage(200, 200, idx1_ref, w1_ref)
    stage(PH, PW, idx2_ref, w2_ref)


def _coords(rois_p):
    grid = RP // 8
    cspec = pl.BlockSpec((1, LANES), lambda i: (0, 0))
    ospec = pl.BlockSpec((8, LANES), lambda i: (i, 0))
    return pl.pallas_call(
        _coords_body,
        grid=(grid,),
        in_specs=[pl.BlockSpec((8, 8), lambda i: (i, 0))] + [cspec] * 4,
        out_specs=[ospec] * 4,
        out_shape=[
            jax.ShapeDtypeStruct((RP, LANES), jnp.int32),
            jax.ShapeDtypeStruct((RP, LANES), jnp.float32),
            jax.ShapeDtypeStruct((RP, LANES), jnp.int32),
            jax.ShapeDtypeStruct((RP, LANES), jnp.float32),
        ],
    )(rois_p, jnp.asarray(_AY), jnp.asarray(_AX), jnp.asarray(_KY),
      jnp.asarray(_KX))


NSUB = 4
SUBROWS = CHUNK_ROWS * 16 // NSUB
NPASS = 4
PC = C // NPASS
TROWS = 2 * 200 * 200


def _gather_sum_cp_kernel(table_hbm, idx_hbm, w_hbm,
                          o0, o1, o2, o3,
                          spm, idx_v, w_v, g_a, g_b, ovm, sem_a, sem_b):
    outs = (o0, o1, o2, o3)
    nc = 2
    hchunks = CHUNKS // 2
    sid = lax.axis_index("s")
    wid = sid * nc + lax.axis_index("c")
    row0 = wid * (CHUNKS * CHUNK_ROWS)
    stage_rows = TROWS // 16

    for p in range(NPASS):
        if p:
            plsc.subcore_barrier()
        pltpu.sync_copy(table_hbm.at[pl.ds(sid * stage_rows, stage_rows),
                                     pl.ds(p * PC, PC)],
                        spm.at[pl.ds(sid * stage_rows, stage_rows)])
        plsc.subcore_barrier()
        out = outs[p]

        def fire(j, g, sem):
            for s in range(NSUB):
                pltpu.async_copy(spm.at[idx_v.at[j * NSUB + s]],
                                 g.at[pl.ds(s * SUBROWS, SUBROWS)], sem)

        def drain(j, g, sem):
            for s in range(NSUB):
                pltpu.make_async_copy(spm.at[idx_v.at[j * NSUB + s]],
                                      g.at[pl.ds(s * SUBROWS, SUBROWS)],
                                      sem).wait()

        def compute(j, g):
            for r in range(CHUNK_ROWS):
                acc = jnp.zeros((16,), jnp.float32)
                wrow = w_v[j, pl.ds(r * 16, 16)]
                for q in range(16):
                    wv = jnp.full((16,), wrow[q], jnp.float32)
                    acc = acc + wv * g[r * 16 + q, :]
                ovm[pl.ds((j * CHUNK_ROWS + r) * PC, PC)] = acc

        for h in range(2):
            pltpu.sync_copy(idx_hbm.at[wid, pl.ds(h * hchunks * NSUB,
                                                  hchunks * NSUB)], idx_v)
            pltpu.sync_copy(w_hbm.at[wid, pl.ds(h * hchunks, hchunks)], w_v)

            fire(0, g_a, sem_a)

            def body(jj, carry):
                j = jj * 2
                fire(j + 1, g_b, sem_b)
                drain(j, g_a, sem_a)
                compute(j, g_a)

                @pl.when(jj < hchunks // 2 - 1)
                def _():
                    fire(j + 2, g_a, sem_a)

                drain(j + 1, g_b, sem_b)
                compute(j + 1, g_b)
                return carry

            lax.fori_loop(0, hchunks // 2, body, 0)
            pltpu.sync_copy(ovm, out.at[pl.ds(
                (row0 + h * hchunks * CHUNK_ROWS) * PC,
                hchunks * CHUNK_ROWS * PC)])


def _gather_sum_cp(table_cp, idx, w):
    mesh = plsc.VectorSubcoreMesh(core_axis_name="c", subcore_axis_name="s")
    oshape = jax.ShapeDtypeStruct((NR * PC,), jnp.float32)
    kfn = functools.partial(
        pl.kernel,
        mesh=mesh,
        compiler_params=pltpu.CompilerParams(use_tc_tiling_on_sc=False),
        out_type=[oshape] * NPASS,
        scratch_types=[
            pltpu.VMEM_SHARED((TROWS, PC), jnp.float32),
            pltpu.VMEM((CHUNKS // 2 * NSUB, SUBROWS), jnp.int32),
            pltpu.VMEM((CHUNKS // 2, 128), jnp.float32),
            pltpu.VMEM((128, PC), jnp.float32),
            pltpu.VMEM((128, PC), jnp.float32),
            pltpu.VMEM((CHUNKS // 2 * CHUNK_ROWS * PC,), jnp.float32),
            pltpu.SemaphoreType.DMA,
            pltpu.SemaphoreType.DMA,
        ],
    )(_gather_sum_cp_kernel)
    return kfn(table_cp, idx, w)


TBL2 = 2 * BINS * PC


def _stage2_kernel(t0, t1, t2, t3, idx_hbm, w_hbm, out_hbm, tb, idx_v, w_v, ovm):
    nc = 2
    hchunks = CHUNKS // 2
    wid = lax.axis_index("s") * nc + lax.axis_index("c")
    for p, tp in enumerate((t0, t1, t2, t3)):
        pltpu.sync_copy(tp.at[pl.ds(0, TBL2)], tb.at[pl.ds(p * TBL2, TBL2)])
    row0 = wid * (CHUNKS * CHUNK_ROWS)

    for h in range(2):
        pltpu.sync_copy(idx_hbm.at[wid, pl.ds(h * hchunks * NSUB,
                                              hchunks * NSUB)], idx_v)
        pltpu.sync_copy(w_hbm.at[wid, pl.ds(h * hchunks, hchunks)], w_v)

        def body(j, carry):
            for r in range(CHUNK_ROWS):
                irow = idx_v[j * NSUB + r // 2, pl.ds((r % 2) * 16, 16)]
                wrow = w_v[j, pl.ds(r * 16, 16)]
                acc = [jnp.zeros((16,), jnp.float32) for _ in range(4)]
                for q in range(16):
                    base = irow[q] * PC
                    wv = jnp.full((16,), wrow[q], jnp.float32)
                    for c in range(4):
                        acc[c] = acc[c] + wv * tb[pl.ds(c * TBL2 + base, 16)]
                for c in range(4):
                    ovm[pl.ds((j * CHUNK_ROWS + r) * C + c * 16, 16)] = acc[c]
            return carry

        lax.fori_loop(0, hchunks, body, 0)
        pltpu.sync_copy(ovm, out_hbm.at[pl.ds(
            (row0 + h * hchunks * CHUNK_ROWS) * C,
            hchunks * CHUNK_ROWS * C)])


def _stage2(parts, idx, w):
    mesh = plsc.VectorSubcoreMesh(core_axis_name="c", subcore_axis_name="s")
    hrows = (CHUNKS // 2) * CHUNK_ROWS
    kfn = functools.partial(
        pl.kernel,
        mesh=mesh,
        compiler_params=pltpu.CompilerParams(use_tc_tiling_on_sc=False),
        out_type=jax.ShapeDtypeStruct((NR * C,), jnp.float32),
        scratch_types=[
            pltpu.VMEM((NPASS * TBL2,), jnp.float32),
            pltpu.VMEM((CHUNKS // 2 * NSUB, SUBROWS), jnp.int32),
            pltpu.VMEM((CHUNKS // 2, 128), jnp.float32),
            pltpu.VMEM((hrows * C,), jnp.float32),
        ],
    )(_stage2_kernel)
    return kfn(*parts, idx, w)


def _pack(a, cols=128):
    return a.reshape(NW, (CHUNKS * 128) // cols, cols)


def kernel(input, rois):
    feat = jnp.transpose(input, (0, 2, 3, 1)).reshape(-1, C)
    rois_p = jnp.zeros((RP, 8), jnp.float32).at[:rois.shape[0], :5].set(rois)
    idx1, w1, idx2, w2 = _coords(rois_p)
    o1_parts = _gather_sum_cp(feat, _pack(idx1, SUBROWS), _pack(w1))
    o2 = _stage2(o1_parts, _pack(idx2, SUBROWS), _pack(w2)).reshape(NR, C)
    out = o2[: rois.shape[0] * BINS].reshape(-1, BINS, C)
    return jnp.transpose(out, (0, 2, 1)).reshape(-1, C, PH, PW)


# speedup vs baseline: 15.0933x; 1.0592x over previous
import functools

import jax
import jax.numpy as jnp
import numpy as np
from jax import lax
from jax.experimental import pallas as pl
from jax.experimental.pallas import tpu as pltpu
from jax.experimental.pallas import tpu_sc as plsc

SCALE = 0.25
PH = PW = 7
S = 2
C = 64
RP = 1024
LANES = 784
BINS = PH * PW
NR = RP * BINS
NW = 32
CHUNK_ROWS = 8
CHUNKS = NR // (NW * CHUNK_ROWS)


def _lane_consts():
    l = np.arange(LANES)
    p = l // 16
    q = l % 16
    py, px = p // PW, p % PW
    ygrp, xgrp = q // 4, q % 4
    sy, ky_i = ygrp // 2, ygrp % 2
    sx, kx_i = xgrp // 2, xgrp % 2
    offy = (sy + 0.5) / S
    offx = (sx + 0.5) / S
    ay = (py + offy).astype(np.float32)
    ax = (px + offx).astype(np.float32)
    ky = ky_i.astype(np.float32)
    kx = kx_i.astype(np.float32)
    return tuple(a.reshape(1, LANES) for a in (ay, ax, ky, kx))


_AY, _AX, _KY, _KX = _lane_consts()


def _coords_body(rois_ref, ay_ref, ax_ref, ky_ref, kx_ref,
                 idx1_ref, w1_ref, idx2_ref, w2_ref):
    r = rois_ref[...]
    b = r[:, 0:1].astype(jnp.int32)
    x1 = r[:, 1:2] * SCALE
    y1 = r[:, 2:3] * SCALE
    x2 = r[:, 3:4] * SCALE
    y2 = r[:, 4:5] * SCALE
    bw = jnp.maximum(x2 - x1, 1.0) * (1.0 / PW)
    bh = jnp.maximum(y2 - y1, 1.0) * (1.0 / PH)
    ay, ax = ay_ref[...], ax_ref[...]
    ky, kx = ky_ref[...], kx_ref[...]
    kyi = ky.astype(jnp.int32)
    kxi = kx.astype(jnp.int32)
    y = y1 + ay * bh
    x = x1 + ax * bw

    def stage(H, W, idx_ref, w_ref):
        yc = jnp.clip(y, 0.0, float(H - 1))
        xc = jnp.clip(x, 0.0, float(W - 1))
        y0f = jnp.floor(yc)
        x0f = jnp.floor(xc)
        ly = yc - y0f
        lx = xc - x0f
        wy = ky * ly + (1.0 - ky) * (1.0 - ly)
        wx = kx * lx + (1.0 - kx) * (1.0 - lx)
        y0 = y0f.astype(jnp.int32)
        x0 = x0f.astype(jnp.int32)
        yi = jnp.minimum(y0 + kyi, H - 1)
        xi = jnp.minimum(x0 + kxi, W - 1)
        idx_ref[...] = b * (H * W) + yi * W + xi
        w_ref[...] = wy * wx * (1.0 / (S * S))

    stage(200, 200, idx1_ref, w1_ref)
    stage(PH, PW, idx2_ref, w2_ref)


def _coords(rois_p):
    grid = RP // 8
    cspec = pl.BlockSpec((1, LANES), lambda i: (0, 0))
    ospec = pl.BlockSpec((8, LANES), lambda i: (i, 0))
    return pl.pallas_call(
        _coords_body,
        grid=(grid,),
        in_specs=[pl.BlockSpec((8, 8), lambda i: (i, 0))] + [cspec] * 4,
        out_specs=[ospec] * 4,
        out_shape=[
            jax.ShapeDtypeStruct((RP, LANES), jnp.int32),
            jax.ShapeDtypeStruct((RP, LANES), jnp.float32),
            jax.ShapeDtypeStruct((RP, LANES), jnp.int32),
            jax.ShapeDtypeStruct((RP, LANES), jnp.float32),
        ],
    )(rois_p, jnp.asarray(_AY), jnp.asarray(_AX), jnp.asarray(_KY),
      jnp.asarray(_KX))


NSUB = 4
SUBROWS = CHUNK_ROWS * 16 // NSUB
NPASS = 4
PC = C // NPASS
TROWS = 2 * 200 * 200


def _gather_sum_cp_kernel(table_hbm, idx_hbm, w_hbm,
                          o0, o1, o2, o3,
                          spm, idx_v, w_v, g_a, g_b, ovm, sem_a, sem_b):
    outs = (o0, o1, o2, o3)
    nc = 2
    hchunks = CHUNKS // 2
    sid = lax.axis_index("s")
    wid = sid * nc + lax.axis_index("c")
    row0 = wid * (CHUNKS * CHUNK_ROWS)
    stage_rows = TROWS // 16

    for p in range(NPASS):
        if p:
            plsc.subcore_barrier()
        pltpu.sync_copy(table_hbm.at[pl.ds(sid * stage_rows, stage_rows),
                                     pl.ds(p * PC, PC)],
                        spm.at[pl.ds(sid * stage_rows, stage_rows)])
        plsc.subcore_barrier()
        out = outs[p]

        def fire(j, g, sem):
            for s in range(NSUB):
                pltpu.async_copy(spm.at[idx_v.at[j * NSUB + s]],
                                 g.at[pl.ds(s * SUBROWS, SUBROWS)], sem)

        def drain(j, g, sem):
            for s in range(NSUB):
                pltpu.make_async_copy(spm.at[idx_v.at[j * NSUB + s]],
                                      g.at[pl.ds(s * SUBROWS, SUBROWS)],
                                      sem).wait()

        def compute(j, g):
            for r in range(CHUNK_ROWS):
                part = [jnp.zeros((16,), jnp.float32) for _ in range(4)]
                wrow = w_v[j, pl.ds(r * 16, 16)]
                for q in range(16):
                    wv = jnp.full((16,), wrow[q], jnp.float32)
                    part[q % 4] = part[q % 4] + wv * g[r * 16 + q, :]
                acc = (part[0] + part[1]) + (part[2] + part[3])
                ovm[pl.ds((j * CHUNK_ROWS + r) * PC, PC)] = acc

        for h in range(2):
            pltpu.sync_copy(idx_hbm.at[wid, pl.ds(h * hchunks * NSUB,
                                                  hchunks * NSUB)], idx_v)
            pltpu.sync_copy(w_hbm.at[wid, pl.ds(h * hchunks, hchunks)], w_v)

            fire(0, g_a, sem_a)

            def body(jj, carry):
                j = jj * 2
                fire(j + 1, g_b, sem_b)
                drain(j, g_a, sem_a)
                compute(j, g_a)

                @pl.when(jj < hchunks // 2 - 1)
                def _():
                    fire(j + 2, g_a, sem_a)

                drain(j + 1, g_b, sem_b)
                compute(j + 1, g_b)
                return carry

            lax.fori_loop(0, hchunks // 2, body, 0)
            pltpu.sync_copy(ovm, out.at[pl.ds(
                (row0 + h * hchunks * CHUNK_ROWS) * PC,
                hchunks * CHUNK_ROWS * PC)])


def _gather_sum_cp(table_cp, idx, w):
    mesh = plsc.VectorSubcoreMesh(core_axis_name="c", subcore_axis_name="s")
    oshape = jax.ShapeDtypeStruct((NR * PC,), jnp.float32)
    kfn = functools.partial(
        pl.kernel,
        mesh=mesh,
        compiler_params=pltpu.CompilerParams(use_tc_tiling_on_sc=False),
        out_type=[oshape] * NPASS,
        scratch_types=[
            pltpu.VMEM_SHARED((TROWS, PC), jnp.float32),
            pltpu.VMEM((CHUNKS // 2 * NSUB, SUBROWS), jnp.int32),
            pltpu.VMEM((CHUNKS // 2, 128), jnp.float32),
            pltpu.VMEM((128, PC), jnp.float32),
            pltpu.VMEM((128, PC), jnp.float32),
            pltpu.VMEM((CHUNKS // 2 * CHUNK_ROWS * PC,), jnp.float32),
            pltpu.SemaphoreType.DMA,
            pltpu.SemaphoreType.DMA,
        ],
    )(_gather_sum_cp_kernel)
    return kfn(table_cp, idx, w)


TBL2 = 2 * BINS * PC


def _stage2_kernel(t0, t1, t2, t3, idx_hbm, w_hbm, out_hbm, tb, idx_v, w_v, ovm):
    nc = 2
    hchunks = CHUNKS // 2
    wid = lax.axis_index("s") * nc + lax.axis_index("c")
    for p, tp in enumerate((t0, t1, t2, t3)):
        pltpu.sync_copy(tp.at[pl.ds(0, TBL2)], tb.at[pl.ds(p * TBL2, TBL2)])
    row0 = wid * (CHUNKS * CHUNK_ROWS)

    for h in range(2):
        pltpu.sync_copy(idx_hbm.at[wid, pl.ds(h * hchunks * NSUB,
                                              hchunks * NSUB)], idx_v)
        pltpu.sync_copy(w_hbm.at[wid, pl.ds(h * hchunks, hchunks)], w_v)

        def body(j, carry):
            for r in range(CHUNK_ROWS):
                irow = idx_v[j * NSUB + r // 2, pl.ds((r % 2) * 16, 16)]
                wrow = w_v[j, pl.ds(r * 16, 16)]
                part = [jnp.zeros((16,), jnp.float32) for _ in range(8)]
                for q in range(16):
                    base = irow[q] * PC
                    wv = jnp.full((16,), wrow[q], jnp.float32)
                    for c in range(4):
                        part[c * 2 + q % 2] = (part[c * 2 + q % 2]
                                               + wv * tb[pl.ds(c * TBL2 + base,
                                                               16)])
                for c in range(4):
                    ovm[pl.ds((j * CHUNK_ROWS + r) * C + c * 16, 16)] = (
                        part[c * 2] + part[c * 2 + 1])
            return carry

        lax.fori_loop(0, hchunks, body, 0)
        pltpu.sync_copy(ovm, out_hbm.at[pl.ds(
            (row0 + h * hchunks * CHUNK_ROWS) * C,
            hchunks * CHUNK_ROWS * C)])


def _stage2(parts, idx, w):
    mesh = plsc.VectorSubcoreMesh(core_axis_name="c", subcore_axis_name="s")
    hrows = (CHUNKS // 2) * CHUNK_ROWS
    kfn = functools.partial(
        pl.kernel,
        mesh=mesh,
        compiler_params=pltpu.CompilerParams(use_tc_tiling_on_sc=False),
        out_type=jax.ShapeDtypeStruct((NR * C,), jnp.float32),
        scratch_types=[
            pltpu.VMEM((NPASS * TBL2,), jnp.float32),
            pltpu.VMEM((CHUNKS // 2 * NSUB, SUBROWS), jnp.int32),
            pltpu.VMEM((CHUNKS // 2, 128), jnp.float32),
            pltpu.VMEM((hrows * C,), jnp.float32),
        ],
    )(_stage2_kernel)
    return kfn(*parts, idx, w)


def _pack(a, cols=128):
    return a.reshape(NW, (CHUNKS * 128) // cols, cols)


def kernel(input, rois):
    feat = jnp.transpose(input, (0, 2, 3, 1)).reshape(-1, C)
    rois_p = jnp.zeros((RP, 8), jnp.float32).at[:rois.shape[0], :5].set(rois)
    idx1, w1, idx2, w2 = _coords(rois_p)
    o1_parts = _gather_sum_cp(feat, _pack(idx1, SUBROWS), _pack(w1))
    o2 = _stage2(o1_parts, _pack(idx2, SUBROWS), _pack(w2)).reshape(NR, C)
    out = o2[: rois.shape[0] * BINS].reshape(-1, BINS, C)
    return jnp.transpose(out, (0, 2, 1)).reshape(-1, C, PH, PW)


# speedup vs baseline: 15.1027x; 1.0006x over previous
import functools

import jax
import jax.numpy as jnp
import numpy as np
from jax import lax
from jax.experimental import pallas as pl
from jax.experimental.pallas import tpu as pltpu
from jax.experimental.pallas import tpu_sc as plsc

SCALE = 0.25
PH = PW = 7
S = 2
C = 64
RP = 1024
LANES = 784
BINS = PH * PW
NR = RP * BINS
NW = 32
CHUNK_ROWS = 8
CHUNKS = NR // (NW * CHUNK_ROWS)


def _lane_consts():
    l = np.arange(LANES)
    p = l // 16
    q = l % 16
    py, px = p // PW, p % PW
    ygrp, xgrp = q // 4, q % 4
    sy, ky_i = ygrp // 2, ygrp % 2
    sx, kx_i = xgrp // 2, xgrp % 2
    offy = (sy + 0.5) / S
    offx = (sx + 0.5) / S
    ay = (py + offy).astype(np.float32)
    ax = (px + offx).astype(np.float32)
    ky = ky_i.astype(np.float32)
    kx = kx_i.astype(np.float32)
    return tuple(a.reshape(1, LANES) for a in (ay, ax, ky, kx))


_AY, _AX, _KY, _KX = _lane_consts()


def _coords_body(rois_ref, ay_ref, ax_ref, ky_ref, kx_ref,
                 idx1_ref, w1_ref, idx2_ref, w2_ref):
    r = rois_ref[...]
    b = r[:, 0:1].astype(jnp.int32)
    x1 = r[:, 1:2] * SCALE
    y1 = r[:, 2:3] * SCALE
    x2 = r[:, 3:4] * SCALE
    y2 = r[:, 4:5] * SCALE
    bw = jnp.maximum(x2 - x1, 1.0) * (1.0 / PW)
    bh = jnp.maximum(y2 - y1, 1.0) * (1.0 / PH)
    ay, ax = ay_ref[...], ax_ref[...]
    ky, kx = ky_ref[...], kx_ref[...]
    kyi = ky.astype(jnp.int32)
    kxi = kx.astype(jnp.int32)
    y = y1 + ay * bh
    x = x1 + ax * bw

    def stage(H, W, idx_ref, w_ref):
        yc = jnp.clip(y, 0.0, float(H - 1))
        xc = jnp.clip(x, 0.0, float(W - 1))
        y0f = jnp.floor(yc)
        x0f = jnp.floor(xc)
        ly = yc - y0f
        lx = xc - x0f
        wy = ky * ly + (1.0 - ky) * (1.0 - ly)
        wx = kx * lx + (1.0 - kx) * (1.0 - lx)
        y0 = y0f.astype(jnp.int32)
        x0 = x0f.astype(jnp.int32)
        yi = jnp.minimum(y0 + kyi, H - 1)
        xi = jnp.minimum(x0 + kxi, W - 1)
        idx_ref[...] = b * (H * W) + yi * W + xi
        w_ref[...] = wy * wx * (1.0 / (S * S))

    stage(200, 200, idx1_ref, w1_ref)
    stage(PH, PW, idx2_ref, w2_ref)


def _coords(rois_p):
    grid = RP // 8
    cspec = pl.BlockSpec((1, LANES), lambda i: (0, 0))
    ospec = pl.BlockSpec((8, LANES), lambda i: (i, 0))
    return pl.pallas_call(
        _coords_body,
        grid=(grid,),
        in_specs=[pl.BlockSpec((8, 8), lambda i: (i, 0))] + [cspec] * 4,
        out_specs=[ospec] * 4,
        out_shape=[
            jax.ShapeDtypeStruct((RP, LANES), jnp.int32),
            jax.ShapeDtypeStruct((RP, LANES), jnp.float32),
            jax.ShapeDtypeStruct((RP, LANES), jnp.int32),
            jax.ShapeDtypeStruct((RP, LANES), jnp.float32),
        ],
    )(rois_p, jnp.asarray(_AY), jnp.asarray(_AX), jnp.asarray(_KY),
      jnp.asarray(_KX))


NSUB = 1
SUBROWS = CHUNK_ROWS * 16 // NSUB
NPASS = 4
PC = C // NPASS
TROWS = 2 * 200 * 200


def _gather_sum_cp_kernel(table_hbm, idx_hbm, w_hbm,
                          o0, o1, o2, o3,
                          spm, idx_v, w_v, g_a, g_b, ovm, sem_a, sem_b):
    outs = (o0, o1, o2, o3)
    nc = 2
    hchunks = CHUNKS // 2
    sid = lax.axis_index("s")
    wid = sid * nc + lax.axis_index("c")
    row0 = wid * (CHUNKS * CHUNK_ROWS)
    stage_rows = TROWS // 16

    for p in range(NPASS):
        if p:
            plsc.subcore_barrier()
        pltpu.sync_copy(table_hbm.at[pl.ds(sid * stage_rows, stage_rows),
                                     pl.ds(p * PC, PC)],
                        spm.at[pl.ds(sid * stage_rows, stage_rows)])
        plsc.subcore_barrier()
        out = outs[p]

        def fire(j, g, sem):
            for s in range(NSUB):
                pltpu.async_copy(spm.at[idx_v.at[j * NSUB + s]],
                                 g.at[pl.ds(s * SUBROWS, SUBROWS)], sem)

        def drain(j, g, sem):
            for s in range(NSUB):
                pltpu.make_async_copy(spm.at[idx_v.at[j * NSUB + s]],
                                      g.at[pl.ds(s * SUBROWS, SUBROWS)],
                                      sem).wait()

        def compute(j, g):
            for r in range(CHUNK_ROWS):
                part = [jnp.zeros((16,), jnp.float32) for _ in range(4)]
                wrow = w_v[j, pl.ds(r * 16, 16)]
                for q in range(16):
                    wv = jnp.full((16,), wrow[q], jnp.float32)
                    part[q % 4] = part[q % 4] + wv * g[r * 16 + q, :]
                acc = (part[0] + part[1]) + (part[2] + part[3])
                ovm[pl.ds((j * CHUNK_ROWS + r) * PC, PC)] = acc

        for h in range(2):
            pltpu.sync_copy(idx_hbm.at[wid, pl.ds(h * hchunks * NSUB,
                                                  hchunks * NSUB)], idx_v)
            pltpu.sync_copy(w_hbm.at[wid, pl.ds(h * hchunks, hchunks)], w_v)

            fire(0, g_a, sem_a)

            def body(jj, carry):
                j = jj * 2
                fire(j + 1, g_b, sem_b)
                drain(j, g_a, sem_a)
                compute(j, g_a)

                @pl.when(jj < hchunks // 2 - 1)
                def _():
                    fire(j + 2, g_a, sem_a)

                drain(j + 1, g_b, sem_b)
                compute(j + 1, g_b)
                return carry

            lax.fori_loop(0, hchunks // 2, body, 0)
            pltpu.sync_copy(ovm, out.at[pl.ds(
                (row0 + h * hchunks * CHUNK_ROWS) * PC,
                hchunks * CHUNK_ROWS * PC)])


def _gather_sum_cp(table_cp, idx, w):
    mesh = plsc.VectorSubcoreMesh(core_axis_name="c", subcore_axis_name="s")
    oshape = jax.ShapeDtypeStruct((NR * PC,), jnp.float32)
    kfn = functools.partial(
        pl.kernel,
        mesh=mesh,
        compiler_params=pltpu.CompilerParams(use_tc_tiling_on_sc=False),
        out_type=[oshape] * NPASS,
        scratch_types=[
            pltpu.VMEM_SHARED((TROWS, PC), jnp.float32),
            pltpu.VMEM((CHUNKS // 2 * NSUB, SUBROWS), jnp.int32),
            pltpu.VMEM((CHUNKS // 2, 128), jnp.float32),
            pltpu.VMEM((128, PC), jnp.float32),
            pltpu.VMEM((128, PC), jnp.float32),
            pltpu.VMEM((CHUNKS // 2 * CHUNK_ROWS * PC,), jnp.float32),
            pltpu.SemaphoreType.DMA,
            pltpu.SemaphoreType.DMA,
        ],
    )(_gather_sum_cp_kernel)
    return kfn(table_cp, idx, w)


TBL2 = 2 * BINS * PC


def _stage2_kernel(t0, t1, t2, t3, idx_hbm, w_hbm, out_hbm, tb, idx_v, w_v, ovm):
    nc = 2
    hchunks = CHUNKS // 2
    wid = lax.axis_index("s") * nc + lax.axis_index("c")
    for p, tp in enumerate((t0, t1, t2, t3)):
        pltpu.sync_copy(tp.at[pl.ds(0, TBL2)], tb.at[pl.ds(p * TBL2, TBL2)])
    row0 = wid * (CHUNKS * CHUNK_ROWS)

    for h in range(2):
        pltpu.sync_copy(idx_hbm.at[wid, pl.ds(h * hchunks * NSUB,
                                              hchunks * NSUB)], idx_v)
        pltpu.sync_copy(w_hbm.at[wid, pl.ds(h * hchunks, hchunks)], w_v)

        def body(j, carry):
            for r in range(CHUNK_ROWS):
                irow = idx_v[j, pl.ds(r * 16, 16)]
                wrow = w_v[j, pl.ds(r * 16, 16)]
                part = [jnp.zeros((16,), jnp.float32) for _ in range(8)]
                for q in range(16):
                    base = irow[q] * PC
                    wv = jnp.full((16,), wrow[q], jnp.float32)
                    for c in range(4):
                        part[c * 2 + q % 2] = (part[c * 2 + q % 2]
                                               + wv * tb[pl.ds(c * TBL2 + base,
                                                               16)])
                for c in range(4):
                    ovm[pl.ds((j * CHUNK_ROWS + r) * C + c * 16, 16)] = (
                        part[c * 2] + part[c * 2 + 1])
            return carry

        lax.fori_loop(0, hchunks, body, 0)
        pltpu.sync_copy(ovm, out_hbm.at[pl.ds(
            (row0 + h * hchunks * CHUNK_ROWS) * C,
            hchunks * CHUNK_ROWS * C)])


def _stage2(parts, idx, w):
    mesh = plsc.VectorSubcoreMesh(core_axis_name="c", subcore_axis_name="s")
    hrows = (CHUNKS // 2) * CHUNK_ROWS
    kfn = functools.partial(
        pl.kernel,
        mesh=mesh,
        compiler_params=pltpu.CompilerParams(use_tc_tiling_on_sc=False),
        out_type=jax.ShapeDtypeStruct((NR * C,), jnp.float32),
        scratch_types=[
            pltpu.VMEM((NPASS * TBL2,), jnp.float32),
            pltpu.VMEM((CHUNKS // 2 * NSUB, SUBROWS), jnp.int32),
            pltpu.VMEM((CHUNKS // 2, 128), jnp.float32),
            pltpu.VMEM((hrows * C,), jnp.float32),
        ],
    )(_stage2_kernel)
    return kfn(*parts, idx, w)


def _pack(a, cols=128):
    return a.reshape(NW, (CHUNKS * 128) // cols, cols)


def kernel(input, rois):
    feat = jnp.transpose(input, (0, 2, 3, 1)).reshape(-1, C)
    rois_p = jnp.zeros((RP, 8), jnp.float32).at[:rois.shape[0], :5].set(rois)
    idx1, w1, idx2, w2 = _coords(rois_p)
    o1_parts = _gather_sum_cp(feat, _pack(idx1, SUBROWS), _pack(w1))
    o2 = _stage2(o1_parts, _pack(idx2, SUBROWS), _pack(w2)).reshape(NR, C)
    out = o2[: rois.shape[0] * BINS].reshape(-1, BINS, C)
    return jnp.transpose(out, (0, 2, 1)).reshape(-1, C, PH, PW)


# speedup vs baseline: 15.1339x; 1.0021x over previous
import functools

import jax
import jax.numpy as jnp
import numpy as np
from jax import lax
from jax.experimental import pallas as pl
from jax.experimental.pallas import tpu as pltpu
from jax.experimental.pallas import tpu_sc as plsc

SCALE = 0.25
PH = PW = 7
S = 2
C = 64
RP = 1024
LANES = 784
BINS = PH * PW
NR = RP * BINS
NW = 32
CHUNK_ROWS = 8
CHUNKS = NR // (NW * CHUNK_ROWS)


def _lane_consts():
    l = np.arange(LANES)
    p = l // 16
    q = l % 16
    py, px = p // PW, p % PW
    ygrp, xgrp = q // 4, q % 4
    sy, ky_i = ygrp // 2, ygrp % 2
    sx, kx_i = xgrp // 2, xgrp % 2
    offy = (sy + 0.5) / S
    offx = (sx + 0.5) / S
    ay = (py + offy).astype(np.float32)
    ax = (px + offx).astype(np.float32)
    ky = ky_i.astype(np.float32)
    kx = kx_i.astype(np.float32)
    return tuple(a.reshape(1, LANES) for a in (ay, ax, ky, kx))


_AY, _AX, _KY, _KX = _lane_consts()


def _coords_body(rois_ref, ay_ref, ax_ref, ky_ref, kx_ref,
                 idx1_ref, w1_ref, idx2_ref, w2_ref):
    r = rois_ref[...]
    b = r[:, 0:1].astype(jnp.int32)
    x1 = r[:, 1:2] * SCALE
    y1 = r[:, 2:3] * SCALE
    x2 = r[:, 3:4] * SCALE
    y2 = r[:, 4:5] * SCALE
    bw = jnp.maximum(x2 - x1, 1.0) * (1.0 / PW)
    bh = jnp.maximum(y2 - y1, 1.0) * (1.0 / PH)
    ay, ax = ay_ref[...], ax_ref[...]
    ky, kx = ky_ref[...], kx_ref[...]
    kyi = ky.astype(jnp.int32)
    kxi = kx.astype(jnp.int32)
    y = y1 + ay * bh
    x = x1 + ax * bw

    def stage(H, W, idx_ref, w_ref):
        yc = jnp.clip(y, 0.0, float(H - 1))
        xc = jnp.clip(x, 0.0, float(W - 1))
        y0f = jnp.floor(yc)
        x0f = jnp.floor(xc)
        ly = yc - y0f
        lx = xc - x0f
        wy = ky * ly + (1.0 - ky) * (1.0 - ly)
        wx = kx * lx + (1.0 - kx) * (1.0 - lx)
        y0 = y0f.astype(jnp.int32)
        x0 = x0f.astype(jnp.int32)
        yi = jnp.minimum(y0 + kyi, H - 1)
        xi = jnp.minimum(x0 + kxi, W - 1)
        idx_ref[...] = b * (H * W) + yi * W + xi
        w_ref[...] = wy * wx * (1.0 / (S * S))

    stage(200, 200, idx1_ref, w1_ref)
    stage(PH, PW, idx2_ref, w2_ref)


def _coords(rois_p):
    grid = RP // 8
    cspec = pl.BlockSpec((1, LANES), lambda i: (0, 0))
    ospec = pl.BlockSpec((8, LANES), lambda i: (i, 0))
    return pl.pallas_call(
        _coords_body,
        grid=(grid,),
        in_specs=[pl.BlockSpec((8, 8), lambda i: (i, 0))] + [cspec] * 4,
        out_specs=[ospec] * 4,
        out_shape=[
            jax.ShapeDtypeStruct((RP, LANES), jnp.int32),
            jax.ShapeDtypeStruct((RP, LANES), jnp.float32),
            jax.ShapeDtypeStruct((RP, LANES), jnp.int32),
            jax.ShapeDtypeStruct((RP, LANES), jnp.float32),
        ],
    )(rois_p, jnp.asarray(_AY), jnp.asarray(_AX), jnp.asarray(_KY),
      jnp.asarray(_KX))


NSUB = 1
SUBROWS = CHUNK_ROWS * 16 // NSUB
NPASS = 4
PC = C // NPASS
TROWS = 2 * 200 * 200


def _gather_sum_cp_kernel(table_hbm, idx_hbm, w_hbm,
                          o0, o1, o2, o3,
                          spm, idx_v, w_v, g_a, g_b, ovm, sem_a, sem_b):
    outs = (o0, o1, o2, o3)
    nc = 2
    hchunks = CHUNKS // 2
    sid = lax.axis_index("s")
    wid = sid * nc + lax.axis_index("c")
    row0 = wid * (CHUNKS * CHUNK_ROWS)
    stage_rows = TROWS // 16

    for p in range(NPASS):
        if p:
            plsc.subcore_barrier()
        with jax.named_scope("stage_table"):
            pltpu.sync_copy(table_hbm.at[pl.ds(sid * stage_rows, stage_rows),
                                         pl.ds(p * PC, PC)],
                            spm.at[pl.ds(sid * stage_rows, stage_rows)])
            plsc.subcore_barrier()
        out = outs[p]

        def fire(j, g, sem):
            for s in range(NSUB):
                pltpu.async_copy(spm.at[idx_v.at[j * NSUB + s]],
                                 g.at[pl.ds(s * SUBROWS, SUBROWS)], sem)

        def drain(j, g, sem):
            for s in range(NSUB):
                pltpu.make_async_copy(spm.at[idx_v.at[j * NSUB + s]],
                                      g.at[pl.ds(s * SUBROWS, SUBROWS)],
                                      sem).wait()

        def compute(j, g):
            for r in range(CHUNK_ROWS):
                part = [jnp.zeros((16,), jnp.float32) for _ in range(4)]
                wrow = w_v[j, pl.ds(r * 16, 16)]
                for q in range(16):
                    wv = jnp.full((16,), wrow[q], jnp.float32)
                    part[q % 4] = part[q % 4] + wv * g[r * 16 + q, :]
                acc = (part[0] + part[1]) + (part[2] + part[3])
                ovm[pl.ds((j * CHUNK_ROWS + r) * PC, PC)] = acc

        for h in range(2):
            with jax.named_scope("slabs"):
                pltpu.sync_copy(idx_hbm.at[wid, pl.ds(h * hchunks * NSUB,
                                                      hchunks * NSUB)], idx_v)
                pltpu.sync_copy(w_hbm.at[wid, pl.ds(h * hchunks, hchunks)],
                                w_v)

            fire(0, g_a, sem_a)

            def body(jj, carry):
                j = jj * 2
                fire(j + 1, g_b, sem_b)
                drain(j, g_a, sem_a)
                compute(j, g_a)

                @pl.when(jj < hchunks // 2 - 1)
                def _():
                    fire(j + 2, g_a, sem_a)

                drain(j + 1, g_b, sem_b)
                compute(j + 1, g_b)
                return carry

            with jax.named_scope("chunks"):
                lax.fori_loop(0, hchunks // 2, body, 0)
            with jax.named_scope("writeback"):
                pltpu.sync_copy(ovm, out.at[pl.ds(
                    (row0 + h * hchunks * CHUNK_ROWS) * PC,
                    hchunks * CHUNK_ROWS * PC)])


def _gather_sum_cp(table_cp, idx, w):
    mesh = plsc.VectorSubcoreMesh(core_axis_name="c", subcore_axis_name="s")
    oshape = jax.ShapeDtypeStruct((NR * PC,), jnp.float32)
    kfn = functools.partial(
        pl.kernel,
        mesh=mesh,
        compiler_params=pltpu.CompilerParams(use_tc_tiling_on_sc=False),
        out_type=[oshape] * NPASS,
        scratch_types=[
            pltpu.VMEM_SHARED((TROWS, PC), jnp.float32),
            pltpu.VMEM((CHUNKS // 2 * NSUB, SUBROWS), jnp.int32),
            pltpu.VMEM((CHUNKS // 2, 128), jnp.float32),
            pltpu.VMEM((128, PC), jnp.float32),
            pltpu.VMEM((128, PC), jnp.float32),
            pltpu.VMEM((CHUNKS // 2 * CHUNK_ROWS * PC,), jnp.float32),
            pltpu.SemaphoreType.DMA,
            pltpu.SemaphoreType.DMA,
        ],
    )(_gather_sum_cp_kernel)
    return kfn(table_cp, idx, w)


TBL2 = 2 * BINS * PC


def _stage2_kernel(t0, t1, t2, t3, idx_hbm, w_hbm, out_hbm, tb, idx_v, w_v, ovm):
    nc = 2
    hchunks = CHUNKS // 2
    wid = lax.axis_index("s") * nc + lax.axis_index("c")
    for p, tp in enumerate((t0, t1, t2, t3)):
        pltpu.sync_copy(tp.at[pl.ds(0, TBL2)], tb.at[pl.ds(p * TBL2, TBL2)])
    row0 = wid * (CHUNKS * CHUNK_ROWS)

    for h in range(2):
        pltpu.sync_copy(idx_hbm.at[wid, pl.ds(h * hchunks * NSUB,
                                              hchunks * NSUB)], idx_v)
        pltpu.sync_copy(w_hbm.at[wid, pl.ds(h * hchunks, hchunks)], w_v)

        def body(j, carry):
            for r in range(CHUNK_ROWS):
                irow = idx_v[j, pl.ds(r * 16, 16)]
                wrow = w_v[j, pl.ds(r * 16, 16)]
                part = [jnp.zeros((16,), jnp.float32) for _ in range(8)]
                for q in range(16):
                    base = irow[q] * PC
                    wv = jnp.full((16,), wrow[q], jnp.float32)
                    for c in range(4):
                        part[c * 2 + q % 2] = (part[c * 2 + q % 2]
                                               + wv * tb[pl.ds(c * TBL2 + base,
                                                               16)])
                for c in range(4):
                    ovm[pl.ds((j * CHUNK_ROWS + r) * C + c * 16, 16)] = (
                        part[c * 2] + part[c * 2 + 1])
            return carry

        lax.fori_loop(0, hchunks, body, 0)
        pltpu.sync_copy(ovm, out_hbm.at[pl.ds(
            (row0 + h * hchunks * CHUNK_ROWS) * C,
            hchunks * CHUNK_ROWS * C)])


def _stage2(parts, idx, w):
    mesh = plsc.VectorSubcoreMesh(core_axis_name="c", subcore_axis_name="s")
    hrows = (CHUNKS // 2) * CHUNK_ROWS
    kfn = functools.partial(
        pl.kernel,
        mesh=mesh,
        compiler_params=pltpu.CompilerParams(use_tc_tiling_on_sc=False),
        out_type=jax.ShapeDtypeStruct((NR * C,), jnp.float32),
        scratch_types=[
            pltpu.VMEM((NPASS * TBL2,), jnp.float32),
            pltpu.VMEM((CHUNKS // 2 * NSUB, SUBROWS), jnp.int32),
            pltpu.VMEM((CHUNKS // 2, 128), jnp.float32),
            pltpu.VMEM((hrows * C,), jnp.float32),
        ],
    )(_stage2_kernel)
    return kfn(*parts, idx, w)


def _pack(a, cols=128):
    return a.reshape(NW, (CHUNKS * 128) // cols, cols)


def kernel(input, rois):
    feat = jnp.transpose(input, (0, 2, 3, 1)).reshape(-1, C)
    rois_p = jnp.zeros((RP, 8), jnp.float32).at[:rois.shape[0], :5].set(rois)
    idx1, w1, idx2, w2 = _coords(rois_p)
    o1_parts = _gather_sum_cp(feat, _pack(idx1, SUBROWS), _pack(w1))
    o2 = _stage2(o1_parts, _pack(idx2, SUBROWS), _pack(w2)).reshape(NR, C)
    out = o2[: rois.shape[0] * BINS].reshape(-1, BINS, C)
    return jnp.transpose(out, (0, 2, 1)).reshape(-1, C, PH, PW)
